# baseline (speedup 1.0000x reference)
import functools

import jax
import jax.numpy as jnp
from jax import lax
from jax.experimental import pallas as pl
from jax.experimental.pallas import tpu as pltpu
from jax.experimental.pallas import tpu_sc as plsc

_N = 10000
_NP = 10240
_E = 320000
_NW = 32
_CH = 80
_EP = _NW * _CH * 128
_ROWS_PER_TILE = _NP // 16

@functools.lru_cache(maxsize=None)
def _get_mesh():
    return plsc.VectorSubcoreMesh(
        core_axis_name="c", subcore_axis_name="s", num_cores=2, num_subcores=16
    )


def _zero_vec_store(ref, n16):
    z = jnp.zeros((16,), jnp.float32)

    def body(j, carry):
        ref[pl.ds(j * 16, 16)] = z
        return carry

    lax.fori_loop(0, n16, body, 0)


@functools.lru_cache(maxsize=None)
def _make_sc_deg():
    @functools.partial(
        pl.kernel,
        out_type=jax.ShapeDtypeStruct((2, _NP), jnp.float32),
        mesh=_get_mesh(),
        scratch_types=[
            pltpu.VMEM((_CH, 128), jnp.int32),
            pltpu.VMEM((128,), jnp.float32),
            pltpu.VMEM((_ROWS_PER_TILE,), jnp.float32),
            pltpu.VMEM_SHARED((_NP,), jnp.float32),
            pltpu.SemaphoreType.DMA,
        ],
    )
    def sc_deg(dst_hbm, deg_out, dst_v, ones_v, buf_v, acc_sh, sem):
        c = lax.axis_index("c")
        s = lax.axis_index("s")
        w = c * 16 + s
        pltpu.sync_copy(dst_hbm.at[pl.ds(w * _CH, _CH)], dst_v)
        one = jnp.ones((16,), jnp.float32)
        for k in range(8):
            ones_v[pl.ds(k * 16, 16)] = one
        _zero_vec_store(buf_v, _ROWS_PER_TILE // 16)
        pltpu.sync_copy(buf_v, acc_sh.at[pl.ds(s * _ROWS_PER_TILE, _ROWS_PER_TILE)])
        plsc.subcore_barrier()

        def body(i, carry):
            pltpu.sync_copy(ones_v, acc_sh.at[dst_v.at[i]], add=True)
            return carry

        lax.fori_loop(0, _CH, body, 0)
        plsc.subcore_barrier()
        pltpu.sync_copy(acc_sh.at[pl.ds(s * _ROWS_PER_TILE, _ROWS_PER_TILE)], buf_v)
        pltpu.sync_copy(buf_v, deg_out.at[c, pl.ds(s * _ROWS_PER_TILE, _ROWS_PER_TILE)])

    return sc_deg


@functools.lru_cache(maxsize=None)
def _make_sc_scatter():
    @functools.partial(
        pl.kernel,
        out_type=jax.ShapeDtypeStruct((2, _NP, 128), jnp.float32),
        mesh=_get_mesh(),
        scratch_types=[
            pltpu.VMEM((2, 16, 128), jnp.int32),
            pltpu.VMEM((2, 16, 128), jnp.int32),
            pltpu.VMEM((2, 128, 128), jnp.float32),
            pltpu.VMEM_SHARED((_NP, 128), jnp.float32),
            pltpu.SemaphoreType.DMA,
            pltpu.SemaphoreType.DMA,
            pltpu.SemaphoreType.DMA,
        ],
    )
    def sc_scatter(
        g_hbm, src_hbm, dst_hbm, out_hbm, src_v, dst_v, rows_v, acc_sh, sem, isem, ssem
    ):
        c = lax.axis_index("c")
        s = lax.axis_index("s")

        z = jnp.zeros((16,), jnp.float32)

        def zbody(j, carry):
            for k in range(8):
                rows_v[0, j, pl.ds(k * 16, 16)] = z
            return carry

        lax.fori_loop(0, 128, zbody, 0)
        for i in range(_ROWS_PER_TILE // 128):
            pltpu.sync_copy(
                rows_v.at[0], acc_sh.at[pl.ds(s * _ROWS_PER_TILE + i * 128, 128)]
            )
        plsc.subcore_barrier()

        band = s * 160 + lax.select(c == 0, 0, 128)
        n_sub = lax.select(c == 0, 8, 2)
        n_ch = n_sub * 16

        pltpu.sync_copy(src_hbm.at[pl.ds(band, 16)], src_v.at[0])
        pltpu.sync_copy(dst_hbm.at[pl.ds(band, 16)], dst_v.at[0])
        pltpu.async_copy(g_hbm.at[src_v.at[0, 0]], rows_v.at[0], sem)

        def body(t, carry2):
            h = t // 16
            r = lax.rem(t, 16)
            hb = lax.rem(h, 2)
            cur = lax.rem(t, 2)

            @pl.when(jnp.logical_and(r == 0, h + 1 < n_sub))
            def _():
                nb = band + (h + 1) * 16
                nhb = lax.rem(h + 1, 2)
                pltpu.async_copy(src_hbm.at[pl.ds(nb, 16)], src_v.at[nhb], isem)
                pltpu.async_copy(dst_hbm.at[pl.ds(nb, 16)], dst_v.at[nhb], isem)

            @pl.when(t >= 1)
            def _():
                pltpu.make_async_copy(
                    rows_v.at[cur], acc_sh.at[dst_v.at[hb, r]], ssem
                ).wait()

            @pl.when(jnp.logical_and(r == 15, t + 1 < n_ch))
            def _():
                nhb = lax.rem(h + 1, 2)
                pltpu.make_async_copy(
                    src_hbm.at[pl.ds(band, 16)], src_v.at[nhb], isem
                ).wait()
                pltpu.make_async_copy(
                    dst_hbm.at[pl.ds(band, 16)], dst_v.at[nhb], isem
                ).wait()
                pltpu.async_copy(
                    g_hbm.at[src_v.at[nhb, 0]], rows_v.at[lax.rem(t + 1, 2)], sem
                )

            @pl.when(jnp.logical_and(r < 15, t + 1 < n_ch))
            def _():
                pltpu.async_copy(
                    g_hbm.at[src_v.at[hb, r + 1]],
                    rows_v.at[lax.rem(t + 1, 2)],
                    sem,
                )

            pltpu.make_async_copy(
                g_hbm.at[src_v.at[hb, r]], rows_v.at[cur], sem
            ).wait()
            pltpu.async_copy(rows_v.at[cur], acc_sh.at[dst_v.at[hb, r]], ssem, add=True)
            return carry2

        lax.fori_loop(0, n_ch, body, 0)
        pltpu.make_async_copy(
            rows_v.at[0], acc_sh.at[dst_v.at[0, 0]], ssem
        ).wait()
        plsc.subcore_barrier()
        for i in range(_ROWS_PER_TILE // 128):
            r0 = s * _ROWS_PER_TILE + i * 128
            pltpu.sync_copy(acc_sh.at[pl.ds(r0, 128)], rows_v.at[0])
            pltpu.sync_copy(rows_v.at[0], out_hbm.at[c, pl.ds(r0, 128)])

    return sc_scatter


_B = 640
_GRID = _NP // _B


def _k1_body(x_ref, w_ref, d0_ref, d1_ref, o_ref):
    dinv = lax.rsqrt(d0_ref[...] + d1_ref[...] + 1.0)
    o_ref[...] = dinv * jnp.dot(
        x_ref[...], w_ref[...], preferred_element_type=jnp.float32
    )


def _k2_body(p0_ref, p1_ref, g_ref, d0_ref, d1_ref, b_ref, w_ref, o_ref):
    dinv = lax.rsqrt(d0_ref[...] + d1_ref[...] + 1.0)
    res = dinv * (p0_ref[...] + p1_ref[...] + g_ref[...]) + b_ref[...]
    o_ref[...] = dinv * jnp.dot(res, w_ref[...], preferred_element_type=jnp.float32)


def _k3_body(p0_ref, p1_ref, g_ref, d0_ref, d1_ref, b_ref, w_ref, b3_ref, o_ref):
    dinv = lax.rsqrt(d0_ref[...] + d1_ref[...] + 1.0)
    res = dinv * (p0_ref[...] + p1_ref[...] + g_ref[...]) + b_ref[...]
    o_ref[...] = (
        jnp.dot(res, w_ref[...], preferred_element_type=jnp.float32) + b3_ref[...]
    )


def _row_spec(d):
    return pl.BlockSpec((_B, d), lambda i: (i, 0))


def _rep_spec(a, b):
    return pl.BlockSpec((a, b), lambda i: (0, 0))


_k1 = pl.pallas_call(
    _k1_body,
    grid=(_GRID,),
    in_specs=[_row_spec(128), _rep_spec(128, 128), _row_spec(1), _row_spec(1)],
    out_specs=_row_spec(128),
    out_shape=jax.ShapeDtypeStruct((_NP, 128), jnp.float32),
)

_k2 = pl.pallas_call(
    _k2_body,
    grid=(_GRID,),
    in_specs=[
        _row_spec(128),
        _row_spec(128),
        _row_spec(128),
        _row_spec(1),
        _row_spec(1),
        _rep_spec(1, 128),
        _rep_spec(128, 128),
    ],
    out_specs=_row_spec(128),
    out_shape=jax.ShapeDtypeStruct((_NP, 128), jnp.float32),
)

_k3 = pl.pallas_call(
    _k3_body,
    grid=(_N // 400,),
    in_specs=[
        pl.BlockSpec((400, 128), lambda i: (i, 0)),
        pl.BlockSpec((400, 128), lambda i: (i, 0)),
        pl.BlockSpec((400, 128), lambda i: (i, 0)),
        pl.BlockSpec((400, 1), lambda i: (i, 0)),
        pl.BlockSpec((400, 1), lambda i: (i, 0)),
        _rep_spec(1, 128),
        _rep_spec(128, 64),
        _rep_spec(1, 64),
    ],
    out_specs=pl.BlockSpec((400, 64), lambda i: (i, 0)),
    out_shape=jax.ShapeDtypeStruct((_N, 64), jnp.float32),
)


def kernel(seq, edge_index, W1, b1, W2, b2, W3, b3):
    src = edge_index[0]
    dst = edge_index[1]
    pad = jnp.full((_EP - _E,), _N, dtype=jnp.int32)
    src2 = jnp.concatenate([src, pad]).reshape(_EP // 128, 128)
    dst2 = jnp.concatenate([dst, pad]).reshape(_EP // 128, 128)
    seq_p = jnp.pad(seq, ((0, _NP - _N), (0, 0)))

    deg_p = _make_sc_deg()(dst2)
    d0 = deg_p[0].reshape(_NP, 1)
    d1 = deg_p[1].reshape(_NP, 1)

    g1 = _k1(seq_p, W1, d0, d1)
    p = _make_sc_scatter()(g1, src2, dst2)
    g2 = _k2(p[0], p[1], g1, d0, d1, b1.reshape(1, 128), W2)
    q = _make_sc_scatter()(g2, src2, dst2)
    return _k3(q[0], q[1], g2, d0, d1, b2.reshape(1, 128), W3.T, b3.reshape(1, 64))


# speedup vs baseline: 1.0145x; 1.0145x over previous
import functools

import jax
import jax.numpy as jnp
from jax import lax
from jax.experimental import pallas as pl
from jax.experimental.pallas import tpu as pltpu
from jax.experimental.pallas import tpu_sc as plsc

_N = 10000
_NP = 10240
_E = 320000
_NW = 32
_CH = 80
_EP = _NW * _CH * 128
_ROWS_PER_TILE = _NP // 16

@functools.lru_cache(maxsize=None)
def _get_mesh():
    return plsc.VectorSubcoreMesh(
        core_axis_name="c", subcore_axis_name="s", num_cores=2, num_subcores=16
    )


def _zero_vec_store(ref, n16):
    z = jnp.zeros((16,), jnp.float32)

    def body(j, carry):
        ref[pl.ds(j * 16, 16)] = z
        return carry

    lax.fori_loop(0, n16, body, 0)


@functools.lru_cache(maxsize=None)
def _make_sc_deg():
    @functools.partial(
        pl.kernel,
        out_type=jax.ShapeDtypeStruct((2, _NP), jnp.float32),
        mesh=_get_mesh(),
        scratch_types=[
            pltpu.VMEM((_CH, 128), jnp.int32),
            pltpu.VMEM((128,), jnp.float32),
            pltpu.VMEM((_ROWS_PER_TILE,), jnp.float32),
            pltpu.VMEM_SHARED((_NP,), jnp.float32),
            pltpu.SemaphoreType.DMA,
        ],
    )
    def sc_deg(dst_hbm, deg_out, dst_v, ones_v, buf_v, acc_sh, sem):
        c = lax.axis_index("c")
        s = lax.axis_index("s")
        w = c * 16 + s
        pltpu.sync_copy(dst_hbm.at[pl.ds(w * _CH, _CH)], dst_v)
        one = jnp.ones((16,), jnp.float32)
        for k in range(8):
            ones_v[pl.ds(k * 16, 16)] = one
        _zero_vec_store(buf_v, _ROWS_PER_TILE // 16)
        pltpu.sync_copy(buf_v, acc_sh.at[pl.ds(s * _ROWS_PER_TILE, _ROWS_PER_TILE)])
        plsc.subcore_barrier()

        def body(i, carry):
            pltpu.sync_copy(ones_v, acc_sh.at[dst_v.at[i]], add=True)
            return carry

        lax.fori_loop(0, _CH, body, 0)
        plsc.subcore_barrier()
        pltpu.sync_copy(acc_sh.at[pl.ds(s * _ROWS_PER_TILE, _ROWS_PER_TILE)], buf_v)
        pltpu.sync_copy(buf_v, deg_out.at[c, pl.ds(s * _ROWS_PER_TILE, _ROWS_PER_TILE)])

    return sc_deg


@functools.lru_cache(maxsize=None)
def _make_sc_scatter():
    @functools.partial(
        pl.kernel,
        out_type=jax.ShapeDtypeStruct((2, _NP, 128), jnp.float32),
        mesh=_get_mesh(),
        scratch_types=[
            pltpu.VMEM((2, 16, 128), jnp.int32),
            pltpu.VMEM((2, 16, 128), jnp.int32),
            pltpu.VMEM((2, 128, 128), jnp.float32),
            pltpu.VMEM_SHARED((_NP, 128), jnp.float32),
            pltpu.SemaphoreType.DMA,
            pltpu.SemaphoreType.DMA,
            pltpu.SemaphoreType.DMA,
        ],
    )
    def sc_scatter(
        g_hbm, src_hbm, dst_hbm, out_hbm, src_v, dst_v, rows_v, acc_sh, sem, isem, ssem
    ):
        c = lax.axis_index("c")
        s = lax.axis_index("s")

        z = jnp.zeros((16,), jnp.float32)

        def zbody(j, carry):
            for k in range(8):
                rows_v[0, j, pl.ds(k * 16, 16)] = z
            return carry

        lax.fori_loop(0, 128, zbody, 0)
        for i in range(_ROWS_PER_TILE // 128):
            pltpu.sync_copy(
                rows_v.at[0], acc_sh.at[pl.ds(s * _ROWS_PER_TILE + i * 128, 128)]
            )
        plsc.subcore_barrier()

        band = s * 160 + lax.select(c == 0, 0, 128)
        n_sub = lax.select(c == 0, 8, 2)
        n_ch = n_sub * 16

        pltpu.sync_copy(src_hbm.at[pl.ds(band, 16)], src_v.at[0])
        pltpu.sync_copy(dst_hbm.at[pl.ds(band, 16)], dst_v.at[0])
        pltpu.async_copy(g_hbm.at[src_v.at[0, 0]], rows_v.at[0], sem)

        def body(t, carry2):
            h = t // 16
            r = lax.rem(t, 16)
            hb = lax.rem(h, 2)
            cur = lax.rem(t, 2)

            @pl.when(jnp.logical_and(r == 0, h + 1 < n_sub))
            def _():
                nb = band + (h + 1) * 16
                nhb = lax.rem(h + 1, 2)
                pltpu.async_copy(src_hbm.at[pl.ds(nb, 16)], src_v.at[nhb], isem)
                pltpu.async_copy(dst_hbm.at[pl.ds(nb, 16)], dst_v.at[nhb], isem)

            @pl.when(t >= 1)
            def _():
                pltpu.make_async_copy(
                    rows_v.at[cur], acc_sh.at[dst_v.at[hb, r]], ssem
                ).wait()

            @pl.when(jnp.logical_and(r == 15, t + 1 < n_ch))
            def _():
                nhb = lax.rem(h + 1, 2)
                pltpu.make_async_copy(
                    src_hbm.at[pl.ds(band, 16)], src_v.at[nhb], isem
                ).wait()
                pltpu.make_async_copy(
                    dst_hbm.at[pl.ds(band, 16)], dst_v.at[nhb], isem
                ).wait()
                pltpu.async_copy(
                    g_hbm.at[src_v.at[nhb, 0]], rows_v.at[lax.rem(t + 1, 2)], sem
                )

            @pl.when(jnp.logical_and(r < 15, t + 1 < n_ch))
            def _():
                pltpu.async_copy(
                    g_hbm.at[src_v.at[hb, r + 1]],
                    rows_v.at[lax.rem(t + 1, 2)],
                    sem,
                )

            pltpu.make_async_copy(
                g_hbm.at[src_v.at[hb, r]], rows_v.at[cur], sem
            ).wait()
            pltpu.async_copy(rows_v.at[cur], acc_sh.at[dst_v.at[hb, r]], ssem, add=True)
            return carry2

        lax.fori_loop(0, n_ch, body, 0)
        pltpu.make_async_copy(
            rows_v.at[0], acc_sh.at[dst_v.at[0, 0]], ssem
        ).wait()
        plsc.subcore_barrier()
        for i in range(_ROWS_PER_TILE // 128):
            r0 = s * _ROWS_PER_TILE + i * 128
            pltpu.sync_copy(acc_sh.at[pl.ds(r0, 128)], rows_v.at[0])
            pltpu.sync_copy(rows_v.at[0], out_hbm.at[c, pl.ds(r0, 128)])

    return sc_scatter


_B = 640
_GRID = _NP // _B


def _k1_body(x_ref, w_ref, d0_ref, d1_ref, o_ref):
    dinv = lax.rsqrt(d0_ref[...] + d1_ref[...] + 1.0)
    o_ref[...] = dinv * jnp.dot(
        x_ref[...], w_ref[...], preferred_element_type=jnp.float32
    )


def _k2_body(p0_ref, p1_ref, g_ref, d0_ref, d1_ref, b_ref, w_ref, o_ref):
    dinv = lax.rsqrt(d0_ref[...] + d1_ref[...] + 1.0)
    res = dinv * (p0_ref[...] + p1_ref[...] + g_ref[...]) + b_ref[...]
    o_ref[...] = dinv * jnp.dot(res, w_ref[...], preferred_element_type=jnp.float32)


def _k3_body(p0_ref, p1_ref, g_ref, d0_ref, d1_ref, b_ref, w_ref, b3_ref, o_ref):
    dinv = lax.rsqrt(d0_ref[...] + d1_ref[...] + 1.0)
    res = dinv * (p0_ref[...] + p1_ref[...] + g_ref[...]) + b_ref[...]
    o_ref[...] = (
        jnp.dot(res, w_ref[...], preferred_element_type=jnp.float32) + b3_ref[...]
    )


def _row_spec(d):
    return pl.BlockSpec((_B, d), lambda i: (i, 0))


def _rep_spec(a, b):
    return pl.BlockSpec((a, b), lambda i: (0, 0))


_k1 = pl.pallas_call(
    _k1_body,
    grid=(_GRID,),
    in_specs=[_row_spec(128), _rep_spec(128, 128), _row_spec(1), _row_spec(1)],
    out_specs=_row_spec(128),
    out_shape=jax.ShapeDtypeStruct((_NP, 128), jnp.float32),
)

_k2 = pl.pallas_call(
    _k2_body,
    grid=(_GRID,),
    in_specs=[
        _row_spec(128),
        _row_spec(128),
        _row_spec(128),
        _row_spec(1),
        _row_spec(1),
        _rep_spec(1, 128),
        _rep_spec(128, 128),
    ],
    out_specs=_row_spec(128),
    out_shape=jax.ShapeDtypeStruct((_NP, 128), jnp.float32),
)

_k3 = pl.pallas_call(
    _k3_body,
    grid=(_GRID,),
    in_specs=[
        _row_spec(128),
        _row_spec(128),
        _row_spec(128),
        _row_spec(1),
        _row_spec(1),
        _rep_spec(1, 128),
        _rep_spec(128, 64),
        _rep_spec(1, 64),
    ],
    out_specs=_row_spec(64),
    out_shape=jax.ShapeDtypeStruct((_NP, 64), jnp.float32),
)


def kernel(seq, edge_index, W1, b1, W2, b2, W3, b3):
    src = edge_index[0]
    dst = edge_index[1]
    pad = jnp.full((_EP - _E,), _N, dtype=jnp.int32)
    src2 = jnp.concatenate([src, pad]).reshape(_EP // 128, 128)
    dst2 = jnp.concatenate([dst, pad]).reshape(_EP // 128, 128)
    seq_p = jnp.pad(seq, ((0, _NP - _N), (0, 0)))

    deg_p = _make_sc_deg()(dst2)
    d0 = deg_p[0].reshape(_NP, 1)
    d1 = deg_p[1].reshape(_NP, 1)

    g1 = _k1(seq_p, W1, d0, d1)
    p = _make_sc_scatter()(g1, src2, dst2)
    g2 = _k2(p[0], p[1], g1, d0, d1, b1.reshape(1, 128), W2)
    q = _make_sc_scatter()(g2, src2, dst2)
    out = _k3(q[0], q[1], g2, d0, d1, b2.reshape(1, 128), W3.T, b3.reshape(1, 64))
    return out[:_N]
